# Initial kernel scaffold; baseline (speedup 1.0000x reference)
import functools

import jax
import jax.numpy as jnp
from jax.experimental import pallas as pl
from jax.experimental.pallas import tpu as pltpu
from jax.experimental.pallas import tpu_sc as plsc

NUM_FIELDS = 26
VOCAB = 100000
EMBED_DIM = 32
BATCH = 4096
TOTAL = BATCH * NUM_FIELDS
WINDOW = 128
NUM_WINDOWS = TOTAL // WINDOW

BB = 512
NUM_BB = BATCH // BB

_VECTOR_MESH = plsc.VectorSubcoreMesh(
    core_axis_name="core", subcore_axis_name="subcore")


def _sc_gather(table, flat_idx):
  @functools.partial(
      pl.kernel,
      out_type=jax.ShapeDtypeStruct((TOTAL, EMBED_DIM), jnp.float32),
      mesh=_VECTOR_MESH,
  )
  def gather_kernel(table_hbm, idx_hbm, out_hbm):
    def body(i_vmem, o_vmem):
      pltpu.sync_copy(table_hbm.at[i_vmem.at[0]], o_vmem)

    pltpu.emit_pipeline(
        body,
        grid=(NUM_WINDOWS,),
        in_specs=[pl.BlockSpec((1, WINDOW), lambda i: (0, i))],
        out_specs=[pl.BlockSpec((WINDOW, EMBED_DIM), lambda i: (i, 0))],
        core_axis_name=("core", "subcore"),
        dimension_semantics=(pltpu.PARALLEL,),
    )(idx_hbm, out_hbm)

  return gather_kernel(table, flat_idx)


def _mlp_body(x_ref, d_ref, w1_ref, b1_ref, w2_ref, b2_ref, w3_ref, b3_ref,
              wo_ref, ww_ref, bias_ref, o_ref):
  prec = jax.lax.Precision.HIGHEST
  x = x_ref[...]
  h = jnp.dot(x, w1_ref[...], preferred_element_type=jnp.float32,
              precision=prec)
  h = jnp.maximum(h + b1_ref[...], 0.0)
  h = jnp.dot(h, w2_ref[...], preferred_element_type=jnp.float32,
              precision=prec)
  h = jnp.maximum(h + b2_ref[...], 0.0)
  h = jnp.dot(h, w3_ref[...], preferred_element_type=jnp.float32,
              precision=prec)
  h = jnp.maximum(h + b3_ref[...], 0.0)
  deep = jnp.sum(h * wo_ref[...], axis=1)
  wide = jnp.sum(d_ref[...] * ww_ref[...], axis=1)
  z = 0.5 * (deep + wide + bias_ref[0, 0])
  o_ref[0, :] = jax.nn.sigmoid(z)


def _tc_mlp(embed, dense, w1t, b1, w2t, b2, w3t, b3, wout_row, wide_row, bias):
  wspec = lambda shape: pl.BlockSpec(shape, lambda i: (0, 0))
  return pl.pallas_call(
      _mlp_body,
      grid=(NUM_BB,),
      in_specs=[
          pl.BlockSpec((BB, NUM_FIELDS * EMBED_DIM), lambda i: (i, 0)),
          pl.BlockSpec((BB, 13), lambda i: (i, 0)),
          wspec(w1t.shape), wspec(b1.shape),
          wspec(w2t.shape), wspec(b2.shape),
          wspec(w3t.shape), wspec(b3.shape),
          wspec(wout_row.shape), wspec(wide_row.shape), wspec(bias.shape),
      ],
      out_specs=pl.BlockSpec((1, BB), lambda i: (i, 0)),
      out_shape=jax.ShapeDtypeStruct((NUM_BB, BB), jnp.float32),
  )(embed, dense, w1t, b1, w2t, b2, w3t, b3, wout_row, wide_row, bias)


def kernel(dense_input, sparse_input, embed_tables, wide_W, wide_b,
           W1, b1, W2, b2, W3, b3, Wout, bout):
  table = embed_tables.reshape(NUM_FIELDS * VOCAB, EMBED_DIM)
  offs = (jnp.arange(NUM_FIELDS, dtype=jnp.int32) * VOCAB)[None, :]
  flat_idx = (sparse_input.astype(jnp.int32) + offs).reshape(1, TOTAL)

  embed = _sc_gather(table, flat_idx).reshape(BATCH, NUM_FIELDS * EMBED_DIM)

  bias = (wide_b[0] + bout[0]).reshape(1, 1)
  out = _tc_mlp(
      embed, dense_input,
      W1.T, b1.reshape(1, -1),
      W2.T, b2.reshape(1, -1),
      W3.T, b3.reshape(1, -1),
      Wout, wide_W, bias,
  )
  return out.reshape(BATCH)


# speedup vs baseline: 2.1175x; 2.1175x over previous
import functools

import jax
import jax.numpy as jnp
from jax.experimental import pallas as pl
from jax.experimental.pallas import tpu as pltpu
from jax.experimental.pallas import tpu_sc as plsc

NUM_FIELDS = 26
VOCAB = 100000
EMBED_DIM = 32
BATCH = 4096
TOTAL = BATCH * NUM_FIELDS
WINDOW = 128
NUM_WINDOWS = TOTAL // WINDOW

BB = 512
NUM_BB = BATCH // BB

_VECTOR_MESH = plsc.VectorSubcoreMesh(
    core_axis_name="core", subcore_axis_name="subcore")


def _sc_gather(table, flat_idx):
  @functools.partial(
      pl.kernel,
      out_type=jax.ShapeDtypeStruct((TOTAL, EMBED_DIM), jnp.float32),
      mesh=_VECTOR_MESH,
      compiler_params=pltpu.CompilerParams(use_tc_tiling_on_sc=False),
  )
  def gather_kernel(table_hbm, idx_hbm, out_hbm):
    def body(i_vmem, o_vmem):
      pltpu.sync_copy(table_hbm.at[i_vmem.at[0]], o_vmem)

    pltpu.emit_pipeline(
        body,
        grid=(NUM_WINDOWS,),
        in_specs=[pl.BlockSpec((1, WINDOW), lambda i: (0, i))],
        out_specs=[pl.BlockSpec((WINDOW, EMBED_DIM), lambda i: (i, 0))],
        core_axis_name=("core", "subcore"),
        dimension_semantics=(pltpu.PARALLEL,),
    )(idx_hbm, out_hbm)

  return gather_kernel(table, flat_idx)


def _mlp_body(x_ref, d_ref, w1_ref, b1_ref, w2_ref, b2_ref, w3_ref, b3_ref,
              wo_ref, ww_ref, bias_ref, o_ref):
  prec = jax.lax.Precision.HIGHEST
  x = x_ref[...]
  h = jnp.dot(x, w1_ref[...], preferred_element_type=jnp.float32,
              precision=prec)
  h = jnp.maximum(h + b1_ref[...], 0.0)
  h = jnp.dot(h, w2_ref[...], preferred_element_type=jnp.float32,
              precision=prec)
  h = jnp.maximum(h + b2_ref[...], 0.0)
  h = jnp.dot(h, w3_ref[...], preferred_element_type=jnp.float32,
              precision=prec)
  h = jnp.maximum(h + b3_ref[...], 0.0)
  deep = jnp.sum(h * wo_ref[...], axis=1)
  wide = jnp.sum(d_ref[...] * ww_ref[...], axis=1)
  z = 0.5 * (deep + wide + bias_ref[0, 0])
  o_ref[0, 0, :] = jax.nn.sigmoid(z)


def _tc_mlp(embed, dense, w1t, b1, w2t, b2, w3t, b3, wout_row, wide_row, bias):
  wspec = lambda shape: pl.BlockSpec(shape, lambda i: (0, 0))
  return pl.pallas_call(
      _mlp_body,
      grid=(NUM_BB,),
      in_specs=[
          pl.BlockSpec((BB, NUM_FIELDS * EMBED_DIM), lambda i: (i, 0)),
          pl.BlockSpec((BB, 13), lambda i: (i, 0)),
          wspec(w1t.shape), wspec(b1.shape),
          wspec(w2t.shape), wspec(b2.shape),
          wspec(w3t.shape), wspec(b3.shape),
          wspec(wout_row.shape), wspec(wide_row.shape), wspec(bias.shape),
      ],
      out_specs=pl.BlockSpec((1, 1, BB), lambda i: (i, 0, 0)),
      out_shape=jax.ShapeDtypeStruct((NUM_BB, 1, BB), jnp.float32),
  )(embed, dense, w1t, b1, w2t, b2, w3t, b3, wout_row, wide_row, bias)


def kernel(dense_input, sparse_input, embed_tables, wide_W, wide_b,
           W1, b1, W2, b2, W3, b3, Wout, bout):
  table = embed_tables.reshape(NUM_FIELDS * VOCAB, EMBED_DIM)
  offs = (jnp.arange(NUM_FIELDS, dtype=jnp.int32) * VOCAB)[None, :]
  flat_idx = (sparse_input.astype(jnp.int32) + offs).reshape(1, TOTAL)

  embed = _sc_gather(table, flat_idx).reshape(BATCH, NUM_FIELDS * EMBED_DIM)

  bias = (wide_b[0] + bout[0]).reshape(1, 1)
  out = _tc_mlp(
      embed, dense_input,
      W1.T, b1.reshape(1, -1),
      W2.T, b2.reshape(1, -1),
      W3.T, b3.reshape(1, -1),
      Wout, wide_W, bias,
  )
  return out.reshape(BATCH)


# speedup vs baseline: 2.6074x; 1.2314x over previous
import functools

import jax
import jax.numpy as jnp
from jax.experimental import pallas as pl
from jax.experimental.pallas import tpu as pltpu
from jax.experimental.pallas import tpu_sc as plsc

NUM_FIELDS = 26
VOCAB = 100000
EMBED_DIM = 32
BATCH = 4096
TOTAL = BATCH * NUM_FIELDS
SEG = 128 // EMBED_DIM
SUPER_ROWS = NUM_FIELDS * VOCAB // SEG
WINDOW = 128
NUM_WINDOWS = TOTAL // WINDOW

XDIM = NUM_FIELDS * 128
LAYER1 = 512

BB = 512
NUM_BB = BATCH // BB

_VECTOR_MESH = plsc.VectorSubcoreMesh(
    core_axis_name="core", subcore_axis_name="subcore")


Q_BLK = 1024
FIELD_Q = 25600
NQ = FIELD_Q // Q_BLK
SUPER_PAD = NUM_FIELDS * FIELD_Q
V_NBLK = -(-VOCAB // Q_BLK)


def _repack_body(x0_ref, x1_ref, x2_ref, x3_ref, out_ref):
  out_ref[...] = jnp.concatenate(
      [x0_ref[0].T, x1_ref[0].T, x2_ref[0].T, x3_ref[0].T], axis=1)


def _tc_repack(bt):
  def vspec(s):
    return pl.BlockSpec(
        (1, EMBED_DIM, Q_BLK),
        lambda f, j, s=s: (f, 0, jnp.minimum(s * NQ + j, V_NBLK - 1)))

  return pl.pallas_call(
      _repack_body,
      grid=(NUM_FIELDS, NQ),
      in_specs=[vspec(0), vspec(1), vspec(2), vspec(3)],
      out_specs=pl.BlockSpec((Q_BLK, 128), lambda f, j: (f * NQ + j, 0)),
      out_shape=jax.ShapeDtypeStruct((SUPER_PAD, 128), jnp.float32),
  )(bt, bt, bt, bt)


def _sc_gather(table128, super_idx):
  @functools.partial(
      pl.kernel,
      out_type=jax.ShapeDtypeStruct((TOTAL, 128), jnp.float32),
      mesh=_VECTOR_MESH,
      compiler_params=pltpu.CompilerParams(use_tc_tiling_on_sc=True),
  )
  def gather_kernel(table_hbm, idx_hbm, out_hbm):
    def body(i_vmem, o_vmem):
      pltpu.sync_copy(table_hbm.at[i_vmem.at[0]], o_vmem)

    pltpu.emit_pipeline(
        body,
        grid=(NUM_WINDOWS,),
        in_specs=[pl.BlockSpec((1, WINDOW), lambda i: (0, i))],
        out_specs=[pl.BlockSpec((WINDOW, 128), lambda i: (i, 0))],
        core_axis_name=("core", "subcore"),
        dimension_semantics=(pltpu.PARALLEL,),
    )(idx_hbm, out_hbm)

  return gather_kernel(table128, super_idx)


def _mlp_body(x_ref, seg_ref, d_ref, w1_ref, b1_ref, w2_ref, b2_ref, w3_ref,
              b3_ref, wo_ref, ww_ref, bias_ref, o_ref):
  prec = jax.lax.Precision.HIGHEST
  seg = seg_ref[...]
  seg_l = jnp.broadcast_to(seg[:, :, None], (BB, NUM_FIELDS, 128))
  seg_l = seg_l.reshape(BB, XDIM)
  lane = jax.lax.broadcasted_iota(jnp.int32, (BB, XDIM), 1)
  want = (lane % 128) // EMBED_DIM
  x = jnp.where(seg_l == want, x_ref[...], 0.0)
  h = jnp.dot(x, w1_ref[...], preferred_element_type=jnp.float32,
              precision=prec)
  h = jnp.maximum(h + b1_ref[...], 0.0)
  h = jnp.dot(h, w2_ref[...], preferred_element_type=jnp.float32,
              precision=prec)
  h = jnp.maximum(h + b2_ref[...], 0.0)
  h = jnp.dot(h, w3_ref[...], preferred_element_type=jnp.float32,
              precision=prec)
  h = jnp.maximum(h + b3_ref[...], 0.0)
  deep = jnp.sum(h * wo_ref[...], axis=1)
  wide = jnp.sum(d_ref[...] * ww_ref[...], axis=1)
  z = 0.5 * (deep + wide + bias_ref[0, 0])
  o_ref[0, 0, :] = jax.nn.sigmoid(z)


def _tc_mlp(xg, seg, dense, w1e, b1, w2t, b2, w3t, b3, wout_row, wide_row,
            bias):
  wspec = lambda shape: pl.BlockSpec(shape, lambda i: (0, 0))
  return pl.pallas_call(
      _mlp_body,
      grid=(NUM_BB,),
      in_specs=[
          pl.BlockSpec((BB, XDIM), lambda i: (i, 0)),
          pl.BlockSpec((BB, NUM_FIELDS), lambda i: (i, 0)),
          pl.BlockSpec((BB, 13), lambda i: (i, 0)),
          wspec(w1e.shape), wspec(b1.shape),
          wspec(w2t.shape), wspec(b2.shape),
          wspec(w3t.shape), wspec(b3.shape),
          wspec(wout_row.shape), wspec(wide_row.shape), wspec(bias.shape),
      ],
      out_specs=pl.BlockSpec((1, 1, BB), lambda i: (i, 0, 0)),
      out_shape=jax.ShapeDtypeStruct((NUM_BB, 1, BB), jnp.float32),
  )(xg, seg, dense, w1e, b1, w2t, b2, w3t, b3, wout_row, wide_row, bias)


def kernel(dense_input, sparse_input, embed_tables, wide_W, wide_b,
           W1, b1, W2, b2, W3, b3, Wout, bout):
  table128 = _tc_repack(jnp.transpose(embed_tables, (0, 2, 1)))
  sp = sparse_input.astype(jnp.int32)
  offs = (jnp.arange(NUM_FIELDS, dtype=jnp.int32) * FIELD_Q)[None, :]
  super_idx = (sp % FIELD_Q + offs).reshape(1, TOTAL)
  seg = sp // FIELD_Q

  xg = _sc_gather(table128, super_idx).reshape(BATCH, XDIM)

  w1t = W1.T
  w1e = jnp.broadcast_to(
      w1t.reshape(NUM_FIELDS, 1, EMBED_DIM, LAYER1),
      (NUM_FIELDS, SEG, EMBED_DIM, LAYER1)).reshape(XDIM, LAYER1)

  bias = (wide_b[0] + bout[0]).reshape(1, 1)
  out = _tc_mlp(
      xg, seg, dense_input,
      w1e, b1.reshape(1, -1),
      W2.T, b2.reshape(1, -1),
      W3.T, b3.reshape(1, -1),
      Wout, wide_W, bias,
  )
  return out.reshape(BATCH)


# speedup vs baseline: 3.5058x; 1.3446x over previous
import functools

import jax
import jax.numpy as jnp
from jax.experimental import pallas as pl
from jax.experimental.pallas import tpu as pltpu
from jax.experimental.pallas import tpu_sc as plsc

NUM_FIELDS = 26
VOCAB = 100000
EMBED_DIM = 32
BATCH = 4096
TOTAL = BATCH * NUM_FIELDS
SEG = 128 // EMBED_DIM
SUPER_ROWS = NUM_FIELDS * VOCAB // SEG
WINDOW = 128
NUM_WINDOWS = TOTAL // WINDOW

XDIM = NUM_FIELDS * 128
LAYER1 = 512

BB = 512
NUM_BB = BATCH // BB

_VECTOR_MESH = plsc.VectorSubcoreMesh(
    core_axis_name="core", subcore_axis_name="subcore")


Q_BLK = 1024
FIELD_Q = 25600
NQ = FIELD_Q // Q_BLK
SUPER_PAD = NUM_FIELDS * FIELD_Q
V_NBLK = -(-VOCAB // Q_BLK)


def _repack_body(x0_ref, x1_ref, x2_ref, x3_ref, out_ref):
  xs = jnp.concatenate(
      [x0_ref[0], x1_ref[0], x2_ref[0], x3_ref[0]], axis=0)
  out_ref[...] = xs.T


def _tc_repack(bt):
  def vspec(s):
    return pl.BlockSpec(
        (1, EMBED_DIM, Q_BLK),
        lambda f, j, s=s: (f, 0, jnp.minimum(s * NQ + j, V_NBLK - 1)))

  return pl.pallas_call(
      _repack_body,
      grid=(NUM_FIELDS, NQ),
      in_specs=[vspec(0), vspec(1), vspec(2), vspec(3)],
      out_specs=pl.BlockSpec((Q_BLK, 128), lambda f, j: (f * NQ + j, 0)),
      out_shape=jax.ShapeDtypeStruct((SUPER_PAD, 128), jnp.float32),
  )(bt, bt, bt, bt)


def _sc_gather(table128, super_idx):
  @functools.partial(
      pl.kernel,
      out_type=jax.ShapeDtypeStruct((TOTAL, 128), jnp.float32),
      mesh=_VECTOR_MESH,
      compiler_params=pltpu.CompilerParams(use_tc_tiling_on_sc=True),
  )
  def gather_kernel(table_hbm, idx_hbm, out_hbm):
    def body(i_vmem, o_vmem):
      pltpu.sync_copy(table_hbm.at[i_vmem.at[0]], o_vmem)

    pltpu.emit_pipeline(
        body,
        grid=(NUM_WINDOWS,),
        in_specs=[pl.BlockSpec((1, WINDOW), lambda i: (0, i))],
        out_specs=[pl.BlockSpec((WINDOW, 128), lambda i: (i, 0))],
        core_axis_name=("core", "subcore"),
        dimension_semantics=(pltpu.PARALLEL,),
    )(idx_hbm, out_hbm)

  return gather_kernel(table128, super_idx)


def _mlp_body(x_ref, seg_ref, d_ref, w1_ref, b1_ref, w2_ref, b2_ref, w3_ref,
              b3_ref, wo_ref, ww_ref, bias_ref, o_ref):
  prec = jax.lax.Precision.HIGHEST
  seg = seg_ref[...]
  seg_l = jnp.broadcast_to(seg[:, :, None], (BB, NUM_FIELDS, 128))
  seg_l = seg_l.reshape(BB, XDIM)
  lane = jax.lax.broadcasted_iota(jnp.int32, (BB, XDIM), 1)
  want = (lane % 128) // EMBED_DIM
  x = jnp.where(seg_l == want, x_ref[...], 0.0)
  h = jnp.dot(x, w1_ref[...], preferred_element_type=jnp.float32,
              precision=prec)
  h = jnp.maximum(h + b1_ref[...], 0.0)
  h = jnp.dot(h, w2_ref[...], preferred_element_type=jnp.float32,
              precision=prec)
  h = jnp.maximum(h + b2_ref[...], 0.0)
  h = jnp.dot(h, w3_ref[...], preferred_element_type=jnp.float32,
              precision=prec)
  h = jnp.maximum(h + b3_ref[...], 0.0)
  deep = jnp.sum(h * wo_ref[...], axis=1)
  wide = jnp.sum(d_ref[...] * ww_ref[...], axis=1)
  z = 0.5 * (deep + wide + bias_ref[0, 0])
  o_ref[0, 0, :] = jax.nn.sigmoid(z)


def _tc_mlp(xg, seg, dense, w1e, b1, w2t, b2, w3t, b3, wout_row, wide_row,
            bias):
  wspec = lambda shape: pl.BlockSpec(shape, lambda i: (0, 0))
  return pl.pallas_call(
      _mlp_body,
      grid=(NUM_BB,),
      in_specs=[
          pl.BlockSpec((BB, XDIM), lambda i: (i, 0)),
          pl.BlockSpec((BB, NUM_FIELDS), lambda i: (i, 0)),
          pl.BlockSpec((BB, 13), lambda i: (i, 0)),
          wspec(w1e.shape), wspec(b1.shape),
          wspec(w2t.shape), wspec(b2.shape),
          wspec(w3t.shape), wspec(b3.shape),
          wspec(wout_row.shape), wspec(wide_row.shape), wspec(bias.shape),
      ],
      out_specs=pl.BlockSpec((1, 1, BB), lambda i: (i, 0, 0)),
      out_shape=jax.ShapeDtypeStruct((NUM_BB, 1, BB), jnp.float32),
  )(xg, seg, dense, w1e, b1, w2t, b2, w3t, b3, wout_row, wide_row, bias)


def kernel(dense_input, sparse_input, embed_tables, wide_W, wide_b,
           W1, b1, W2, b2, W3, b3, Wout, bout):
  table128 = _tc_repack(jnp.transpose(embed_tables, (0, 2, 1)))
  sp = sparse_input.astype(jnp.int32)
  offs = (jnp.arange(NUM_FIELDS, dtype=jnp.int32) * FIELD_Q)[None, :]
  super_idx = (sp % FIELD_Q + offs).reshape(1, TOTAL)
  seg = sp // FIELD_Q

  xg = _sc_gather(table128, super_idx).reshape(BATCH, XDIM)

  w1t = W1.T
  w1e = jnp.broadcast_to(
      w1t.reshape(NUM_FIELDS, 1, EMBED_DIM, LAYER1),
      (NUM_FIELDS, SEG, EMBED_DIM, LAYER1)).reshape(XDIM, LAYER1)

  bias = (wide_b[0] + bout[0]).reshape(1, 1)
  out = _tc_mlp(
      xg, seg, dense_input,
      w1e, b1.reshape(1, -1),
      W2.T, b2.reshape(1, -1),
      W3.T, b3.reshape(1, -1),
      Wout, wide_W, bias,
  )
  return out.reshape(BATCH)


# speedup vs baseline: 4.6629x; 1.3300x over previous
import functools

import jax
import jax.numpy as jnp
from jax.experimental import pallas as pl
from jax.experimental.pallas import tpu as pltpu
from jax.experimental.pallas import tpu_sc as plsc

NUM_FIELDS = 26
VOCAB = 100000
EMBED_DIM = 32
BATCH = 4096
TOTAL = BATCH * NUM_FIELDS
SEG = 128 // EMBED_DIM
SUPER_ROWS = NUM_FIELDS * VOCAB // SEG
WINDOW = 128
NUM_WINDOWS = TOTAL // WINDOW

XDIM = NUM_FIELDS * 128
LAYER1 = 512

BB = 512
NUM_BB = BATCH // BB

_VECTOR_MESH = plsc.VectorSubcoreMesh(
    core_axis_name="core", subcore_axis_name="subcore")


Q_BLK = 2560
FIELD_Q = 25600
NQ = FIELD_Q // Q_BLK
SUPER_PAD = NUM_FIELDS * FIELD_Q
V_NBLK = -(-VOCAB // Q_BLK)


def _repack_body(x0_ref, x1_ref, x2_ref, x3_ref, out_ref):
  xs = jnp.concatenate(
      [x0_ref[0], x1_ref[0], x2_ref[0], x3_ref[0]], axis=0)
  out_ref[...] = xs.T


def _tc_repack(bt):
  def vspec(s):
    return pl.BlockSpec(
        (1, EMBED_DIM, Q_BLK),
        lambda f, j, s=s: (f, 0, jnp.minimum(s * NQ + j, V_NBLK - 1)))

  return pl.pallas_call(
      _repack_body,
      grid=(NUM_FIELDS, NQ),
      in_specs=[vspec(0), vspec(1), vspec(2), vspec(3)],
      out_specs=pl.BlockSpec((Q_BLK, 128), lambda f, j: (f * NQ + j, 0)),
      out_shape=jax.ShapeDtypeStruct((SUPER_PAD, 128), jnp.float32),
  )(bt, bt, bt, bt)


def _sc_gather(table128, super_idx):
  @functools.partial(
      pl.kernel,
      out_type=jax.ShapeDtypeStruct((TOTAL, 128), jnp.float32),
      mesh=_VECTOR_MESH,
      compiler_params=pltpu.CompilerParams(use_tc_tiling_on_sc=True),
  )
  def gather_kernel(table_hbm, idx_hbm, out_hbm):
    def body(i_vmem, o_vmem):
      pltpu.sync_copy(table_hbm.at[i_vmem.at[0]], o_vmem)

    pltpu.emit_pipeline(
        body,
        grid=(NUM_WINDOWS,),
        in_specs=[pl.BlockSpec((1, WINDOW), lambda i: (0, i))],
        out_specs=[pl.BlockSpec((WINDOW, 128), lambda i: (i, 0))],
        core_axis_name=("core", "subcore"),
        dimension_semantics=(pltpu.PARALLEL,),
    )(idx_hbm, out_hbm)

  return gather_kernel(table128, super_idx)


def _mlp_body(x_ref, seg_ref, d_ref, w1_ref, b1_ref, w2_ref, b2_ref, w3_ref,
              b3_ref, wo_ref, ww_ref, bias_ref, o_ref):
  prec = jax.lax.Precision.HIGHEST
  seg = seg_ref[...]
  seg_l = jnp.broadcast_to(seg[:, :, None], (BB, NUM_FIELDS, 128))
  seg_l = seg_l.reshape(BB, XDIM)
  lane = jax.lax.broadcasted_iota(jnp.int32, (BB, XDIM), 1)
  want = (lane % 128) // EMBED_DIM
  x = jnp.where(seg_l == want, x_ref[...], 0.0)
  h = jnp.dot(x, w1_ref[...], preferred_element_type=jnp.float32,
              precision=prec)
  h = jnp.maximum(h + b1_ref[...], 0.0)
  h = jnp.dot(h, w2_ref[...], preferred_element_type=jnp.float32,
              precision=prec)
  h = jnp.maximum(h + b2_ref[...], 0.0)
  h = jnp.dot(h, w3_ref[...], preferred_element_type=jnp.float32,
              precision=prec)
  h = jnp.maximum(h + b3_ref[...], 0.0)
  deep = jnp.sum(h * wo_ref[...], axis=1)
  wide = jnp.sum(d_ref[...] * ww_ref[...], axis=1)
  z = 0.5 * (deep + wide + bias_ref[0, 0])
  o_ref[0, 0, :] = jax.nn.sigmoid(z)


def _tc_mlp(xg, seg, dense, w1e, b1, w2t, b2, w3t, b3, wout_row, wide_row,
            bias):
  wspec = lambda shape: pl.BlockSpec(shape, lambda i: (0, 0))
  return pl.pallas_call(
      _mlp_body,
      grid=(NUM_BB,),
      in_specs=[
          pl.BlockSpec((BB, XDIM), lambda i: (i, 0)),
          pl.BlockSpec((BB, NUM_FIELDS), lambda i: (i, 0)),
          pl.BlockSpec((BB, 13), lambda i: (i, 0)),
          wspec(w1e.shape), wspec(b1.shape),
          wspec(w2t.shape), wspec(b2.shape),
          wspec(w3t.shape), wspec(b3.shape),
          wspec(wout_row.shape), wspec(wide_row.shape), wspec(bias.shape),
      ],
      out_specs=pl.BlockSpec((1, 1, BB), lambda i: (i, 0, 0)),
      out_shape=jax.ShapeDtypeStruct((NUM_BB, 1, BB), jnp.float32),
  )(xg, seg, dense, w1e, b1, w2t, b2, w3t, b3, wout_row, wide_row, bias)


def kernel(dense_input, sparse_input, embed_tables, wide_W, wide_b,
           W1, b1, W2, b2, W3, b3, Wout, bout):
  table128 = _tc_repack(jnp.transpose(embed_tables, (0, 2, 1)))
  sp = sparse_input.astype(jnp.int32)
  offs = (jnp.arange(NUM_FIELDS, dtype=jnp.int32) * FIELD_Q)[None, :]
  super_idx = (sp % FIELD_Q + offs).reshape(1, TOTAL)
  seg = sp // FIELD_Q

  xg = _sc_gather(table128, super_idx).reshape(BATCH, XDIM)

  w1t = W1.T
  w1e = jnp.broadcast_to(
      w1t.reshape(NUM_FIELDS, 1, EMBED_DIM, LAYER1),
      (NUM_FIELDS, SEG, EMBED_DIM, LAYER1)).reshape(XDIM, LAYER1)

  bias = (wide_b[0] + bout[0]).reshape(1, 1)
  out = _tc_mlp(
      xg, seg, dense_input,
      w1e, b1.reshape(1, -1),
      W2.T, b2.reshape(1, -1),
      W3.T, b3.reshape(1, -1),
      Wout, wide_W, bias,
  )
  return out.reshape(BATCH)


# speedup vs baseline: 5.4498x; 1.1688x over previous
import functools

import jax
import jax.numpy as jnp
from jax.experimental import pallas as pl
from jax.experimental.pallas import tpu as pltpu
from jax.experimental.pallas import tpu_sc as plsc

NUM_FIELDS = 26
VOCAB = 100000
EMBED_DIM = 32
BATCH = 4096
TOTAL = BATCH * NUM_FIELDS
SEG = 128 // EMBED_DIM
SUPER_ROWS = NUM_FIELDS * VOCAB // SEG
WINDOW = 128
NUM_WINDOWS = TOTAL // WINDOW

XDIM = NUM_FIELDS * 128
LAYER1 = 512

BB = 512
NUM_BB = BATCH // BB

_VECTOR_MESH = plsc.VectorSubcoreMesh(
    core_axis_name="core", subcore_axis_name="subcore")


Q_BLK = 2560
FIELD_Q = 25600
NQ = FIELD_Q // Q_BLK
SUPER_PAD = NUM_FIELDS * FIELD_Q
V_NBLK = -(-VOCAB // Q_BLK)


def _repack_body(x0_ref, x1_ref, x2_ref, x3_ref, out_ref):
  xs = jnp.concatenate(
      [x0_ref[0], x1_ref[0], x2_ref[0], x3_ref[0]], axis=0)
  out_ref[...] = xs.T


def _tc_repack(bt):
  def vspec(s):
    return pl.BlockSpec(
        (1, EMBED_DIM, Q_BLK),
        lambda f, j, s=s: (f, 0, jnp.minimum(s * NQ + j, V_NBLK - 1)))

  return pl.pallas_call(
      _repack_body,
      grid=(NUM_FIELDS, NQ),
      in_specs=[vspec(0), vspec(1), vspec(2), vspec(3)],
      out_specs=pl.BlockSpec((Q_BLK, 128), lambda f, j: (f * NQ + j, 0)),
      out_shape=jax.ShapeDtypeStruct((SUPER_PAD, 128), jnp.float32),
  )(bt, bt, bt, bt)


def _sc_gather(table128, super_idx):
  @functools.partial(
      pl.kernel,
      out_type=jax.ShapeDtypeStruct((TOTAL, 128), jnp.float32),
      mesh=_VECTOR_MESH,
      compiler_params=pltpu.CompilerParams(use_tc_tiling_on_sc=True),
  )
  def gather_kernel(table_hbm, idx_hbm, out_hbm):
    def body(i_vmem, o_vmem):
      pltpu.sync_copy(table_hbm.at[i_vmem.at[0]], o_vmem)

    pltpu.emit_pipeline(
        body,
        grid=(NUM_WINDOWS,),
        in_specs=[pl.BlockSpec((1, WINDOW), lambda i: (0, i))],
        out_specs=[pl.BlockSpec((WINDOW, 128), lambda i: (i, 0))],
        core_axis_name=("core", "subcore"),
        dimension_semantics=(pltpu.PARALLEL,),
    )(idx_hbm, out_hbm)

  return gather_kernel(table128, super_idx)


def _mlp_body(x_ref, seg_ref, d_ref, w1_ref, b1_ref, w2_ref, b2_ref, w3_ref,
              b3_ref, wo_ref, ww_ref, bias_ref, o_ref):
  prec = jax.lax.Precision.DEFAULT
  seg = seg_ref[...]
  seg_l = jnp.broadcast_to(seg[:, :, None], (BB, NUM_FIELDS, 128))
  seg_l = seg_l.reshape(BB, XDIM)
  lane = jax.lax.broadcasted_iota(jnp.int32, (BB, XDIM), 1)
  want = (lane % 128) // EMBED_DIM
  x = jnp.where(seg_l == want, x_ref[...], 0.0)
  h = jnp.dot(x, w1_ref[...], preferred_element_type=jnp.float32,
              precision=prec)
  h = jnp.maximum(h + b1_ref[...], 0.0)
  h = jnp.dot(h, w2_ref[...], preferred_element_type=jnp.float32,
              precision=prec)
  h = jnp.maximum(h + b2_ref[...], 0.0)
  h = jnp.dot(h, w3_ref[...], preferred_element_type=jnp.float32,
              precision=prec)
  h = jnp.maximum(h + b3_ref[...], 0.0)
  deep = jnp.sum(h * wo_ref[...], axis=1)
  wide = jnp.sum(d_ref[...] * ww_ref[...], axis=1)
  z = 0.5 * (deep + wide + bias_ref[0, 0])
  o_ref[0, 0, :] = jax.nn.sigmoid(z)


def _tc_mlp(xg, seg, dense, w1e, b1, w2t, b2, w3t, b3, wout_row, wide_row,
            bias):
  wspec = lambda shape: pl.BlockSpec(shape, lambda i: (0, 0))
  return pl.pallas_call(
      _mlp_body,
      grid=(NUM_BB,),
      in_specs=[
          pl.BlockSpec((BB, XDIM), lambda i: (i, 0)),
          pl.BlockSpec((BB, NUM_FIELDS), lambda i: (i, 0)),
          pl.BlockSpec((BB, 13), lambda i: (i, 0)),
          wspec(w1e.shape), wspec(b1.shape),
          wspec(w2t.shape), wspec(b2.shape),
          wspec(w3t.shape), wspec(b3.shape),
          wspec(wout_row.shape), wspec(wide_row.shape), wspec(bias.shape),
      ],
      out_specs=pl.BlockSpec((1, 1, BB), lambda i: (i, 0, 0)),
      out_shape=jax.ShapeDtypeStruct((NUM_BB, 1, BB), jnp.float32),
  )(xg, seg, dense, w1e, b1, w2t, b2, w3t, b3, wout_row, wide_row, bias)


def kernel(dense_input, sparse_input, embed_tables, wide_W, wide_b,
           W1, b1, W2, b2, W3, b3, Wout, bout):
  table128 = _tc_repack(jnp.transpose(embed_tables, (0, 2, 1)))
  sp = sparse_input.astype(jnp.int32)
  offs = (jnp.arange(NUM_FIELDS, dtype=jnp.int32) * FIELD_Q)[None, :]
  super_idx = (sp % FIELD_Q + offs).reshape(1, TOTAL)
  seg = sp // FIELD_Q

  xg = _sc_gather(table128, super_idx).reshape(BATCH, XDIM)

  w1t = W1.T
  w1e = jnp.broadcast_to(
      w1t.reshape(NUM_FIELDS, 1, EMBED_DIM, LAYER1),
      (NUM_FIELDS, SEG, EMBED_DIM, LAYER1)).reshape(XDIM, LAYER1)

  bias = (wide_b[0] + bout[0]).reshape(1, 1)
  out = _tc_mlp(
      xg, seg, dense_input,
      w1e, b1.reshape(1, -1),
      W2.T, b2.reshape(1, -1),
      W3.T, b3.reshape(1, -1),
      Wout, wide_W, bias,
  )
  return out.reshape(BATCH)


# speedup vs baseline: 6.5720x; 1.2059x over previous
import functools

import jax
import jax.numpy as jnp
from jax.experimental import pallas as pl
from jax.experimental.pallas import tpu as pltpu
from jax.experimental.pallas import tpu_sc as plsc

NUM_FIELDS = 26
VOCAB = 100000
EMBED_DIM = 32
BATCH = 4096
TOTAL = BATCH * NUM_FIELDS
SEG = 128 // EMBED_DIM
SUPER_ROWS = NUM_FIELDS * VOCAB // SEG
WINDOW = 128
NUM_WINDOWS = TOTAL // WINDOW

XDIM = NUM_FIELDS * 128
LAYER1 = 512

BB = 512
NUM_BB = BATCH // BB

_VECTOR_MESH = plsc.VectorSubcoreMesh(
    core_axis_name="core", subcore_axis_name="subcore")


Q_BLK = 5120
FIELD_Q = 25600
NQ = FIELD_Q // Q_BLK
SUPER_PAD = NUM_FIELDS * FIELD_Q
V_NBLK = -(-VOCAB // Q_BLK)


def _repack_body(x0_ref, x1_ref, x2_ref, x3_ref, out_ref):
  xs = jnp.concatenate(
      [x0_ref[0], x1_ref[0], x2_ref[0], x3_ref[0]], axis=0)
  out_ref[...] = xs.T


def _tc_repack(bt):
  def vspec(s):
    return pl.BlockSpec(
        (1, EMBED_DIM, Q_BLK),
        lambda f, j, s=s: (f, 0, jnp.minimum(s * NQ + j, V_NBLK - 1)))

  return pl.pallas_call(
      _repack_body,
      grid=(NUM_FIELDS, NQ),
      in_specs=[vspec(0), vspec(1), vspec(2), vspec(3)],
      out_specs=pl.BlockSpec((Q_BLK, 128), lambda f, j: (f * NQ + j, 0)),
      out_shape=jax.ShapeDtypeStruct((SUPER_PAD, 128), jnp.float32),
  )(bt, bt, bt, bt)


def _sc_gather(table128, super_idx):
  @functools.partial(
      pl.kernel,
      out_type=jax.ShapeDtypeStruct((TOTAL, 128), jnp.float32),
      mesh=_VECTOR_MESH,
      compiler_params=pltpu.CompilerParams(use_tc_tiling_on_sc=True),
  )
  def gather_kernel(table_hbm, idx_hbm, out_hbm):
    def body(i_vmem, o_vmem):
      pltpu.sync_copy(table_hbm.at[i_vmem.at[0]], o_vmem)

    pltpu.emit_pipeline(
        body,
        grid=(NUM_WINDOWS,),
        in_specs=[pl.BlockSpec((1, WINDOW), lambda i: (0, i))],
        out_specs=[pl.BlockSpec((WINDOW, 128), lambda i: (i, 0))],
        core_axis_name=("core", "subcore"),
        dimension_semantics=(pltpu.PARALLEL,),
    )(idx_hbm, out_hbm)

  return gather_kernel(table128, super_idx)


def _mlp_body(x_ref, seg_ref, d_ref, w1_ref, b1_ref, w2_ref, b2_ref, w3_ref,
              b3_ref, wo_ref, ww_ref, bias_ref, o_ref):
  prec = jax.lax.Precision.DEFAULT
  seg = seg_ref[...]
  seg_l = jnp.broadcast_to(seg[:, :, None], (BB, NUM_FIELDS, 128))
  seg_l = seg_l.reshape(BB, XDIM)
  lane = jax.lax.broadcasted_iota(jnp.int32, (BB, XDIM), 1)
  want = (lane % 128) // EMBED_DIM
  x = jnp.where(seg_l == want, x_ref[...], 0.0)
  h = jnp.dot(x, w1_ref[...], preferred_element_type=jnp.float32,
              precision=prec)
  h = jnp.maximum(h + b1_ref[...], 0.0)
  h = jnp.dot(h, w2_ref[...], preferred_element_type=jnp.float32,
              precision=prec)
  h = jnp.maximum(h + b2_ref[...], 0.0)
  h = jnp.dot(h, w3_ref[...], preferred_element_type=jnp.float32,
              precision=prec)
  h = jnp.maximum(h + b3_ref[...], 0.0)
  deep = jnp.sum(h * wo_ref[...], axis=1)
  wide = jnp.sum(d_ref[...] * ww_ref[...], axis=1)
  z = 0.5 * (deep + wide + bias_ref[0, 0])
  o_ref[0, 0, :] = jax.nn.sigmoid(z)


def _tc_mlp(xg, seg, dense, w1e, b1, w2t, b2, w3t, b3, wout_row, wide_row,
            bias):
  wspec = lambda shape: pl.BlockSpec(shape, lambda i: (0, 0))
  return pl.pallas_call(
      _mlp_body,
      grid=(NUM_BB,),
      in_specs=[
          pl.BlockSpec((BB, XDIM), lambda i: (i, 0)),
          pl.BlockSpec((BB, NUM_FIELDS), lambda i: (i, 0)),
          pl.BlockSpec((BB, 13), lambda i: (i, 0)),
          wspec(w1e.shape), wspec(b1.shape),
          wspec(w2t.shape), wspec(b2.shape),
          wspec(w3t.shape), wspec(b3.shape),
          wspec(wout_row.shape), wspec(wide_row.shape), wspec(bias.shape),
      ],
      out_specs=pl.BlockSpec((1, 1, BB), lambda i: (i, 0, 0)),
      out_shape=jax.ShapeDtypeStruct((NUM_BB, 1, BB), jnp.float32),
  )(xg, seg, dense, w1e, b1, w2t, b2, w3t, b3, wout_row, wide_row, bias)


def kernel(dense_input, sparse_input, embed_tables, wide_W, wide_b,
           W1, b1, W2, b2, W3, b3, Wout, bout):
  table128 = _tc_repack(jnp.transpose(embed_tables, (0, 2, 1)))
  sp = sparse_input.astype(jnp.int32)
  offs = (jnp.arange(NUM_FIELDS, dtype=jnp.int32) * FIELD_Q)[None, :]
  super_idx = (sp % FIELD_Q + offs).reshape(1, TOTAL)
  seg = sp // FIELD_Q

  xg = _sc_gather(table128, super_idx).reshape(BATCH, XDIM)

  w1t = W1.T
  w1e = jnp.broadcast_to(
      w1t.reshape(NUM_FIELDS, 1, EMBED_DIM, LAYER1),
      (NUM_FIELDS, SEG, EMBED_DIM, LAYER1)).reshape(XDIM, LAYER1)

  bias = (wide_b[0] + bout[0]).reshape(1, 1)
  out = _tc_mlp(
      xg, seg, dense_input,
      w1e, b1.reshape(1, -1),
      W2.T, b2.reshape(1, -1),
      W3.T, b3.reshape(1, -1),
      Wout, wide_W, bias,
  )
  return out.reshape(BATCH)


# speedup vs baseline: 6.9910x; 1.0638x over previous
import functools

import jax
import jax.numpy as jnp
from jax.experimental import pallas as pl
from jax.experimental.pallas import tpu as pltpu
from jax.experimental.pallas import tpu_sc as plsc

NUM_FIELDS = 26
VOCAB = 100000
EMBED_DIM = 32
BATCH = 4096
TOTAL = BATCH * NUM_FIELDS
SEG = 128 // EMBED_DIM
SUPER_ROWS = NUM_FIELDS * VOCAB // SEG
WINDOW = 128
NUM_WINDOWS = TOTAL // WINDOW

XDIM = NUM_FIELDS * 128
LAYER1 = 512

BB = 512
NUM_BB = BATCH // BB

_VECTOR_MESH = plsc.VectorSubcoreMesh(
    core_axis_name="core", subcore_axis_name="subcore")


Q_BLK = 12800
FIELD_Q = 25600
NQ = FIELD_Q // Q_BLK
SUPER_PAD = NUM_FIELDS * FIELD_Q
V_NBLK = -(-VOCAB // Q_BLK)


def _repack_body(x0_ref, x1_ref, x2_ref, x3_ref, out_ref):
  xs = jnp.concatenate(
      [x0_ref[0], x1_ref[0], x2_ref[0], x3_ref[0]], axis=0)
  out_ref[...] = xs.T


def _tc_repack(bt):
  def vspec(s):
    return pl.BlockSpec(
        (1, EMBED_DIM, Q_BLK),
        lambda f, j, s=s: (f, 0, jnp.minimum(s * NQ + j, V_NBLK - 1)))

  return pl.pallas_call(
      _repack_body,
      grid=(NUM_FIELDS, NQ),
      in_specs=[vspec(0), vspec(1), vspec(2), vspec(3)],
      out_specs=pl.BlockSpec((Q_BLK, 128), lambda f, j: (f * NQ + j, 0)),
      out_shape=jax.ShapeDtypeStruct((SUPER_PAD, 128), jnp.float32),
  )(bt, bt, bt, bt)


def _sc_gather(table128, super_idx):
  @functools.partial(
      pl.kernel,
      out_type=jax.ShapeDtypeStruct((TOTAL, 128), jnp.float32),
      mesh=_VECTOR_MESH,
      compiler_params=pltpu.CompilerParams(use_tc_tiling_on_sc=True),
  )
  def gather_kernel(table_hbm, idx_hbm, out_hbm):
    def body(i_vmem, o_vmem):
      pltpu.sync_copy(table_hbm.at[i_vmem.at[0]], o_vmem)

    pltpu.emit_pipeline(
        body,
        grid=(NUM_WINDOWS,),
        in_specs=[pl.BlockSpec((1, WINDOW), lambda i: (0, i))],
        out_specs=[pl.BlockSpec((WINDOW, 128), lambda i: (i, 0))],
        core_axis_name=("core", "subcore"),
        dimension_semantics=(pltpu.PARALLEL,),
    )(idx_hbm, out_hbm)

  return gather_kernel(table128, super_idx)


def _mlp_body(x_ref, seg_ref, d_ref, w1_ref, b1_ref, w2_ref, b2_ref, w3_ref,
              b3_ref, wo_ref, ww_ref, bias_ref, o_ref):
  prec = jax.lax.Precision.DEFAULT
  seg = seg_ref[...]
  seg_l = jnp.broadcast_to(seg[:, :, None], (BB, NUM_FIELDS, 128))
  seg_l = seg_l.reshape(BB, XDIM)
  lane = jax.lax.broadcasted_iota(jnp.int32, (BB, XDIM), 1)
  want = (lane % 128) // EMBED_DIM
  x = jnp.where(seg_l == want, x_ref[...], 0.0)
  h = jnp.dot(x, w1_ref[...], preferred_element_type=jnp.float32,
              precision=prec)
  h = jnp.maximum(h + b1_ref[...], 0.0)
  h = jnp.dot(h, w2_ref[...], preferred_element_type=jnp.float32,
              precision=prec)
  h = jnp.maximum(h + b2_ref[...], 0.0)
  h = jnp.dot(h, w3_ref[...], preferred_element_type=jnp.float32,
              precision=prec)
  h = jnp.maximum(h + b3_ref[...], 0.0)
  deep = jnp.sum(h * wo_ref[...], axis=1)
  wide = jnp.sum(d_ref[...] * ww_ref[...], axis=1)
  z = 0.5 * (deep + wide + bias_ref[0, 0])
  o_ref[0, 0, :] = jax.nn.sigmoid(z)


def _tc_mlp(xg, seg, dense, w1e, b1, w2t, b2, w3t, b3, wout_row, wide_row,
            bias):
  wspec = lambda shape: pl.BlockSpec(shape, lambda i: (0, 0))
  return pl.pallas_call(
      _mlp_body,
      grid=(NUM_BB,),
      in_specs=[
          pl.BlockSpec((BB, XDIM), lambda i: (i, 0)),
          pl.BlockSpec((BB, NUM_FIELDS), lambda i: (i, 0)),
          pl.BlockSpec((BB, 13), lambda i: (i, 0)),
          wspec(w1e.shape), wspec(b1.shape),
          wspec(w2t.shape), wspec(b2.shape),
          wspec(w3t.shape), wspec(b3.shape),
          wspec(wout_row.shape), wspec(wide_row.shape), wspec(bias.shape),
      ],
      out_specs=pl.BlockSpec((1, 1, BB), lambda i: (i, 0, 0)),
      out_shape=jax.ShapeDtypeStruct((NUM_BB, 1, BB), jnp.float32),
  )(xg, seg, dense, w1e, b1, w2t, b2, w3t, b3, wout_row, wide_row, bias)


def kernel(dense_input, sparse_input, embed_tables, wide_W, wide_b,
           W1, b1, W2, b2, W3, b3, Wout, bout):
  table128 = _tc_repack(jnp.transpose(embed_tables, (0, 2, 1)))
  sp = sparse_input.astype(jnp.int32)
  offs = (jnp.arange(NUM_FIELDS, dtype=jnp.int32) * FIELD_Q)[None, :]
  super_idx = (sp % FIELD_Q + offs).reshape(1, TOTAL)
  seg = sp // FIELD_Q

  xg = _sc_gather(table128, super_idx).reshape(BATCH, XDIM)

  w1t = W1.T
  w1e = jnp.broadcast_to(
      w1t.reshape(NUM_FIELDS, 1, EMBED_DIM, LAYER1),
      (NUM_FIELDS, SEG, EMBED_DIM, LAYER1)).reshape(XDIM, LAYER1)

  bias = (wide_b[0] + bout[0]).reshape(1, 1)
  out = _tc_mlp(
      xg, seg, dense_input,
      w1e, b1.reshape(1, -1),
      W2.T, b2.reshape(1, -1),
      W3.T, b3.reshape(1, -1),
      Wout, wide_W, bias,
  )
  return out.reshape(BATCH)


# speedup vs baseline: 7.1678x; 1.0253x over previous
import functools

import jax
import jax.numpy as jnp
from jax.experimental import pallas as pl
from jax.experimental.pallas import tpu as pltpu
from jax.experimental.pallas import tpu_sc as plsc

NUM_FIELDS = 26
VOCAB = 100000
EMBED_DIM = 32
BATCH = 4096
SEG = 128 // EMBED_DIM
WINDOW = 128

NGROUPS = 2
GF = NUM_FIELDS // NGROUPS
TOTAL_G = BATCH * GF
XG = GF * 128
XDIM = NUM_FIELDS * 128
LAYER1 = 512

BB = 512
NUM_BB = BATCH // BB

Q_BLK = 12800
FIELD_Q = 25600
NQ = FIELD_Q // Q_BLK
GROUP_Q = GF * FIELD_Q
V_NBLK = -(-VOCAB // Q_BLK)

_VECTOR_MESH = plsc.VectorSubcoreMesh(
    core_axis_name="core", subcore_axis_name="subcore")


def _repack_body(x0_ref, x1_ref, x2_ref, x3_ref, out_ref):
  xs = jnp.concatenate(
      [x0_ref[0], x1_ref[0], x2_ref[0], x3_ref[0]], axis=0)
  out_ref[...] = xs.T


def _tc_repack(bt, g):
  def vspec(s):
    return pl.BlockSpec(
        (1, EMBED_DIM, Q_BLK),
        lambda f, j, s=s: (g * GF + f, 0, jnp.minimum(s * NQ + j, V_NBLK - 1)))

  return pl.pallas_call(
      _repack_body,
      grid=(GF, NQ),
      in_specs=[vspec(0), vspec(1), vspec(2), vspec(3)],
      out_specs=pl.BlockSpec((Q_BLK, 128), lambda f, j: (f * NQ + j, 0)),
      out_shape=jax.ShapeDtypeStruct((GROUP_Q, 128), jnp.float32),
  )(bt, bt, bt, bt)


def _sc_gather(table128, super_idx):
  @functools.partial(
      pl.kernel,
      out_type=jax.ShapeDtypeStruct((TOTAL_G, 128), jnp.float32),
      mesh=_VECTOR_MESH,
      compiler_params=pltpu.CompilerParams(use_tc_tiling_on_sc=True),
  )
  def gather_kernel(table_hbm, idx_hbm, out_hbm):
    def body(i_vmem, o_vmem):
      pltpu.sync_copy(table_hbm.at[i_vmem.at[0]], o_vmem)

    pltpu.emit_pipeline(
        body,
        grid=(TOTAL_G // WINDOW,),
        in_specs=[pl.BlockSpec((1, WINDOW), lambda i: (0, i))],
        out_specs=[pl.BlockSpec((WINDOW, 128), lambda i: (i, 0))],
        core_axis_name=("core", "subcore"),
        dimension_semantics=(pltpu.PARALLEL,),
    )(idx_hbm, out_hbm)

  return gather_kernel(table128, super_idx)


def _mask(x, seg, width):
  bb, gf = seg.shape
  seg_l = jnp.broadcast_to(seg[:, :, None], (bb, gf, 128)).reshape(bb, width)
  lane = jax.lax.broadcasted_iota(jnp.int32, (bb, width), 1)
  return jnp.where(seg_l == (lane % 128) // EMBED_DIM, x, 0.0)


def _mlp_body(x0_ref, x1_ref, seg_ref, d_ref, w1a_ref, w1b_ref, b1_ref,
              w2_ref, b2_ref, w3_ref, b3_ref, wo_ref, ww_ref, bias_ref,
              o_ref):
  prec = jax.lax.Precision.DEFAULT
  seg = seg_ref[...]
  x0 = _mask(x0_ref[...], seg[:, :GF], XG)
  x1 = _mask(x1_ref[...], seg[:, GF:], XG)
  h = (jnp.dot(x0, w1a_ref[...], preferred_element_type=jnp.float32,
               precision=prec)
       + jnp.dot(x1, w1b_ref[...], preferred_element_type=jnp.float32,
                 precision=prec))
  h = jnp.maximum(h + b1_ref[...], 0.0)
  h = jnp.dot(h, w2_ref[...], preferred_element_type=jnp.float32,
              precision=prec)
  h = jnp.maximum(h + b2_ref[...], 0.0)
  h = jnp.dot(h, w3_ref[...], preferred_element_type=jnp.float32,
              precision=prec)
  h = jnp.maximum(h + b3_ref[...], 0.0)
  deep = jnp.sum(h * wo_ref[...], axis=1)
  wide = jnp.sum(d_ref[...] * ww_ref[...], axis=1)
  z = 0.5 * (deep + wide + bias_ref[0, 0])
  o_ref[0, 0, :] = jax.nn.sigmoid(z)


def _tc_mlp(x0, x1, seg, dense, w1a, w1b, b1, w2t, b2, w3t, b3, wout_row,
            wide_row, bias):
  wspec = lambda shape: pl.BlockSpec(shape, lambda i: (0, 0))
  return pl.pallas_call(
      _mlp_body,
      grid=(NUM_BB,),
      in_specs=[
          pl.BlockSpec((BB, XG), lambda i: (i, 0)),
          pl.BlockSpec((BB, XG), lambda i: (i, 0)),
          pl.BlockSpec((BB, NUM_FIELDS), lambda i: (i, 0)),
          pl.BlockSpec((BB, 13), lambda i: (i, 0)),
          wspec(w1a.shape), wspec(w1b.shape), wspec(b1.shape),
          wspec(w2t.shape), wspec(b2.shape),
          wspec(w3t.shape), wspec(b3.shape),
          wspec(wout_row.shape), wspec(wide_row.shape), wspec(bias.shape),
      ],
      out_specs=pl.BlockSpec((1, 1, BB), lambda i: (i, 0, 0)),
      out_shape=jax.ShapeDtypeStruct((NUM_BB, 1, BB), jnp.float32),
  )(x0, x1, seg, dense, w1a, w1b, b1, w2t, b2, w3t, b3, wout_row, wide_row,
    bias)


def kernel(dense_input, sparse_input, embed_tables, wide_W, wide_b,
           W1, b1, W2, b2, W3, b3, Wout, bout):
  bt = jnp.transpose(embed_tables, (0, 2, 1))
  sp = sparse_input.astype(jnp.int32)
  offs = (jnp.arange(GF, dtype=jnp.int32) * FIELD_Q)[None, :]
  seg = sp // FIELD_Q

  xs = []
  for g in range(NGROUPS):
    table_g = _tc_repack(bt, g)
    sp_g = sp[:, g * GF:(g + 1) * GF]
    idx_g = (sp_g % FIELD_Q + offs).reshape(1, TOTAL_G)
    xs.append(_sc_gather(table_g, idx_g).reshape(BATCH, XG))

  w1t = W1.T
  w1e = jnp.broadcast_to(
      w1t.reshape(NUM_FIELDS, 1, EMBED_DIM, LAYER1),
      (NUM_FIELDS, SEG, EMBED_DIM, LAYER1)).reshape(XDIM, LAYER1)

  bias = (wide_b[0] + bout[0]).reshape(1, 1)
  out = _tc_mlp(
      xs[0], xs[1], seg, dense_input,
      w1e[:XG], w1e[XG:], b1.reshape(1, -1),
      W2.T, b2.reshape(1, -1),
      W3.T, b3.reshape(1, -1),
      Wout, wide_W, bias,
  )
  return out.reshape(BATCH)


# speedup vs baseline: 8.2355x; 1.1490x over previous
import functools

import jax
import jax.numpy as jnp
from jax.experimental import pallas as pl
from jax.experimental.pallas import tpu as pltpu
from jax.experimental.pallas import tpu_sc as plsc

NUM_FIELDS = 26
VOCAB = 100000
EMBED_DIM = 32
BATCH = 4096
SEG = 128 // EMBED_DIM
WINDOW = 128

NGROUPS = 2
GF = NUM_FIELDS // NGROUPS
TOTAL_G = BATCH * GF
XG = GF * 128
XDIM = NUM_FIELDS * 128
LAYER1 = 512

BB = 512
NUM_BB = BATCH // BB

Q_BLK = 12800
FIELD_Q = 25600
NQ = FIELD_Q // Q_BLK
GROUP_Q = GF * FIELD_Q
V_NBLK = -(-VOCAB // Q_BLK)

_VECTOR_MESH = plsc.VectorSubcoreMesh(
    core_axis_name="core", subcore_axis_name="subcore")


def _repack_body(x0_ref, x1_ref, x2_ref, x3_ref, out_ref):
  xs = jnp.concatenate(
      [x0_ref[0], x1_ref[0], x2_ref[0], x3_ref[0]], axis=0)
  out_ref[...] = xs.T


def _tc_repack(bt, g):
  def vspec(s):
    return pl.BlockSpec(
        (1, EMBED_DIM, Q_BLK),
        lambda f, j, s=s: (g * GF + f, 0, jnp.minimum(s * NQ + j, V_NBLK - 1)))

  return pl.pallas_call(
      _repack_body,
      grid=(GF, NQ),
      in_specs=[vspec(0), vspec(1), vspec(2), vspec(3)],
      out_specs=pl.BlockSpec((Q_BLK, 128), lambda f, j: (f * NQ + j, 0)),
      out_shape=jax.ShapeDtypeStruct((GROUP_Q, 128), jnp.float32),
  )(bt, bt, bt, bt)


def _sc_gather(table128, super_idx):
  nb = BATCH // WINDOW

  @functools.partial(
      pl.kernel,
      out_type=jax.ShapeDtypeStruct((GF, BATCH, 128), jnp.float32),
      mesh=_VECTOR_MESH,
      compiler_params=pltpu.CompilerParams(use_tc_tiling_on_sc=True),
  )
  def gather_kernel(table_hbm, idx_hbm, out_hbm):
    def body(i_vmem, o_vmem):
      pltpu.sync_copy(table_hbm.at[i_vmem.at[0]], o_vmem.at[0])

    pltpu.emit_pipeline(
        body,
        grid=(TOTAL_G // WINDOW,),
        in_specs=[pl.BlockSpec((1, WINDOW), lambda i: (0, i))],
        out_specs=[pl.BlockSpec(
            (1, WINDOW, 128), lambda i: (i // nb, i % nb, 0))],
        core_axis_name=("core", "subcore"),
        dimension_semantics=(pltpu.PARALLEL,),
    )(idx_hbm, out_hbm)

  return gather_kernel(table128, super_idx)


def _mlp_body(x0_ref, x1_ref, seg_ref, d_ref, w1_ref, b1_ref,
              w2_ref, b2_ref, w3_ref, b3_ref, wo_ref, ww_ref, bias_ref,
              o_ref):
  prec = jax.lax.Precision.DEFAULT
  seg = seg_ref[...]
  lane_seg = jax.lax.broadcasted_iota(jnp.int32, (BB, 128), 1) // EMBED_DIM
  h = jnp.zeros((BB, LAYER1), jnp.float32)
  for g, x_ref in ((0, x0_ref), (1, x1_ref)):
    for f in range(GF):
      fg = g * GF + f
      xf = jnp.where(seg[:, fg][:, None] == lane_seg, x_ref[f], 0.0)
      h = h + jnp.dot(xf, w1_ref[pl.ds(fg * 128, 128), :],
                      preferred_element_type=jnp.float32, precision=prec)
  h = jnp.maximum(h + b1_ref[...], 0.0)
  h = jnp.dot(h, w2_ref[...], preferred_element_type=jnp.float32,
              precision=prec)
  h = jnp.maximum(h + b2_ref[...], 0.0)
  h = jnp.dot(h, w3_ref[...], preferred_element_type=jnp.float32,
              precision=prec)
  h = jnp.maximum(h + b3_ref[...], 0.0)
  deep = jnp.sum(h * wo_ref[...], axis=1)
  wide = jnp.sum(d_ref[...] * ww_ref[...], axis=1)
  z = 0.5 * (deep + wide + bias_ref[0, 0])
  o_ref[0, 0, :] = jax.nn.sigmoid(z)


def _tc_mlp(x0, x1, seg, dense, w1e, b1, w2t, b2, w3t, b3, wout_row,
            wide_row, bias):
  wspec = lambda shape: pl.BlockSpec(shape, lambda i: tuple(0 for _ in shape))
  xspec = pl.BlockSpec((GF, BB, 128), lambda i: (0, i, 0))
  return pl.pallas_call(
      _mlp_body,
      grid=(NUM_BB,),
      in_specs=[
          xspec, xspec,
          pl.BlockSpec((BB, NUM_FIELDS), lambda i: (i, 0)),
          pl.BlockSpec((BB, 13), lambda i: (i, 0)),
          wspec(w1e.shape), wspec(b1.shape),
          wspec(w2t.shape), wspec(b2.shape),
          wspec(w3t.shape), wspec(b3.shape),
          wspec(wout_row.shape), wspec(wide_row.shape), wspec(bias.shape),
      ],
      out_specs=pl.BlockSpec((1, 1, BB), lambda i: (i, 0, 0)),
      out_shape=jax.ShapeDtypeStruct((NUM_BB, 1, BB), jnp.float32),
  )(x0, x1, seg, dense, w1e, b1, w2t, b2, w3t, b3, wout_row, wide_row, bias)


def kernel(dense_input, sparse_input, embed_tables, wide_W, wide_b,
           W1, b1, W2, b2, W3, b3, Wout, bout):
  bt = jnp.transpose(embed_tables, (0, 2, 1))
  sp = sparse_input.astype(jnp.int32)
  spt = sp.T
  offs = (jnp.arange(GF, dtype=jnp.int32) * FIELD_Q)[:, None]
  seg = sp // FIELD_Q

  xs = []
  for g in range(NGROUPS):
    table_g = _tc_repack(bt, g)
    spt_g = spt[g * GF:(g + 1) * GF]
    idx_g = (spt_g % FIELD_Q + offs).reshape(1, TOTAL_G)
    xs.append(_sc_gather(table_g, idx_g))

  w1t = W1.T
  w1e = jnp.broadcast_to(
      w1t.reshape(NUM_FIELDS, 1, EMBED_DIM, LAYER1),
      (NUM_FIELDS, SEG, EMBED_DIM, LAYER1)).reshape(XDIM, LAYER1)

  bias = (wide_b[0] + bout[0]).reshape(1, 1)
  out = _tc_mlp(
      xs[0], xs[1], seg, dense_input,
      w1e, b1.reshape(1, -1),
      W2.T, b2.reshape(1, -1),
      W3.T, b3.reshape(1, -1),
      Wout, wide_W, bias,
  )
  return out.reshape(BATCH)


# speedup vs baseline: 9.2026x; 1.1174x over previous
import functools

import jax
import jax.numpy as jnp
from jax.experimental import pallas as pl
from jax.experimental.pallas import tpu as pltpu
from jax.experimental.pallas import tpu_sc as plsc

NUM_FIELDS = 26
VOCAB = 100000
EMBED_DIM = 32
BATCH = 4096
SEG = 128 // EMBED_DIM
WINDOW = 128

NGROUPS = 2
GF = NUM_FIELDS // NGROUPS
TOTAL_G = BATCH * GF
XG = GF * 128
XDIM = NUM_FIELDS * 128
LAYER1 = 512

BB = 512
NUM_BB = BATCH // BB

Q_BLK = 12800
FIELD_Q = 25600
NQ = FIELD_Q // Q_BLK
GROUP_Q = GF * FIELD_Q
V_NBLK = -(-VOCAB // Q_BLK)
HALF = Q_BLK // 2
PHY_F = FIELD_Q // 2

_VECTOR_MESH = plsc.VectorSubcoreMesh(
    core_axis_name="core", subcore_axis_name="subcore")


def _repack_body(x0_ref, x1_ref, x2_ref, x3_ref, out_ref):
  xs = jnp.concatenate(
      [x0_ref[0], x1_ref[0], x2_ref[0], x3_ref[0]], axis=0)
  t = xs.T
  bits = jax.lax.bitcast_convert_type(t, jnp.int32)
  rb = bits + 0x7FFF + (jax.lax.shift_right_logical(bits, 16) & 1)
  hi16 = jax.lax.shift_right_logical(rb, 16)
  half = Q_BLK // 2
  out_ref[...] = jax.lax.shift_left(hi16[:half], 16) | hi16[half:]


def _tc_repack(bt, g):
  def vspec(s):
    return pl.BlockSpec(
        (1, EMBED_DIM, Q_BLK),
        lambda f, j, s=s: (g * GF + f, 0, jnp.minimum(s * NQ + j, V_NBLK - 1)))

  return pl.pallas_call(
      _repack_body,
      grid=(GF, NQ),
      in_specs=[vspec(0), vspec(1), vspec(2), vspec(3)],
      out_specs=pl.BlockSpec((Q_BLK // 2, 128), lambda f, j: (f * NQ + j, 0)),
      out_shape=jax.ShapeDtypeStruct((GROUP_Q // 2, 128), jnp.int32),
  )(bt, bt, bt, bt)


def _sc_gather(table128, super_idx):
  nb = BATCH // WINDOW

  @functools.partial(
      pl.kernel,
      out_type=jax.ShapeDtypeStruct((GF, BATCH, 128), jnp.int32),
      mesh=_VECTOR_MESH,
      compiler_params=pltpu.CompilerParams(use_tc_tiling_on_sc=True),
  )
  def gather_kernel(table_hbm, idx_hbm, out_hbm):
    def body(i_vmem, o_vmem):
      pltpu.sync_copy(table_hbm.at[i_vmem.at[0]], o_vmem.at[0])

    pltpu.emit_pipeline(
        body,
        grid=(TOTAL_G // WINDOW,),
        in_specs=[pl.BlockSpec((1, WINDOW), lambda i: (0, i))],
        out_specs=[pl.BlockSpec(
            (1, WINDOW, 128), lambda i: (i // nb, i % nb, 0))],
        core_axis_name=("core", "subcore"),
        dimension_semantics=(pltpu.PARALLEL,),
    )(idx_hbm, out_hbm)

  return gather_kernel(table128, super_idx)


def _mlp_body(x0_ref, x1_ref, seg_ref, d_ref, w1_ref, b1_ref,
              w2_ref, b2_ref, w3_ref, b3_ref, wo_ref, ww_ref, bias_ref,
              o_ref):
  prec = jax.lax.Precision.DEFAULT
  seg = seg_ref[...]
  lane_seg = jax.lax.broadcasted_iota(jnp.int32, (BB, 256), 1) // EMBED_DIM
  h = jnp.zeros((BB, LAYER1), jnp.float32)
  for g, x_ref in ((0, x0_ref), (1, x1_ref)):
    for f in range(GF):
      fg = g * GF + f
      xi = x_ref[f]
      hi = jax.lax.bitcast_convert_type(xi & jnp.int32(-65536), jnp.float32)
      lo = jax.lax.bitcast_convert_type(
          jax.lax.shift_left(xi, 16), jnp.float32)
      xf = jnp.concatenate([hi, lo], axis=1).astype(jnp.bfloat16)
      xf = jnp.where(seg[:, fg][:, None] == lane_seg, xf, jnp.bfloat16(0.0))
      h = h + jnp.dot(xf, w1_ref[pl.ds(fg * 256, 256), :],
                      preferred_element_type=jnp.float32, precision=prec)
  h = jnp.maximum(h + b1_ref[...], 0.0)
  h = jnp.dot(h, w2_ref[...], preferred_element_type=jnp.float32,
              precision=prec)
  h = jnp.maximum(h + b2_ref[...], 0.0)
  h = jnp.dot(h, w3_ref[...], preferred_element_type=jnp.float32,
              precision=prec)
  h = jnp.maximum(h + b3_ref[...], 0.0)
  deep = jnp.sum(h * wo_ref[...], axis=1)
  wide = jnp.sum(d_ref[...] * ww_ref[...], axis=1)
  z = 0.5 * (deep + wide + bias_ref[0, 0])
  o_ref[0, 0, :] = jax.nn.sigmoid(z)


def _tc_mlp(x0, x1, seg, dense, w1e, b1, w2t, b2, w3t, b3, wout_row,
            wide_row, bias):
  wspec = lambda shape: pl.BlockSpec(shape, lambda i: tuple(0 for _ in shape))
  xspec = pl.BlockSpec((GF, BB, 128), lambda i: (0, i, 0))
  return pl.pallas_call(
      _mlp_body,
      grid=(NUM_BB,),
      in_specs=[
          xspec, xspec,
          pl.BlockSpec((BB, NUM_FIELDS), lambda i: (i, 0)),
          pl.BlockSpec((BB, 13), lambda i: (i, 0)),
          wspec(w1e.shape), wspec(b1.shape),
          wspec(w2t.shape), wspec(b2.shape),
          wspec(w3t.shape), wspec(b3.shape),
          wspec(wout_row.shape), wspec(wide_row.shape), wspec(bias.shape),
      ],
      out_specs=pl.BlockSpec((1, 1, BB), lambda i: (i, 0, 0)),
      out_shape=jax.ShapeDtypeStruct((NUM_BB, 1, BB), jnp.float32),
  )(x0, x1, seg, dense, w1e, b1, w2t, b2, w3t, b3, wout_row, wide_row, bias)


def kernel(dense_input, sparse_input, embed_tables, wide_W, wide_b,
           W1, b1, W2, b2, W3, b3, Wout, bout):
  bt = jnp.transpose(embed_tables, (0, 2, 1))
  sp = sparse_input.astype(jnp.int32)
  spt = sp.T
  offs = (jnp.arange(GF, dtype=jnp.int32) * PHY_F)[:, None]
  kk = sp % FIELD_Q
  seg = (kk % Q_BLK) // HALF * SEG + sp // FIELD_Q
  kt = spt % FIELD_Q
  row_t = (kt // Q_BLK) * HALF + (kt % Q_BLK) % HALF

  xs = []
  for g in range(NGROUPS):
    table_g = _tc_repack(bt, g)
    idx_g = (row_t[g * GF:(g + 1) * GF] + offs).reshape(1, TOTAL_G)
    xs.append(_sc_gather(table_g, idx_g))

  w1t = W1.T.astype(jnp.bfloat16)
  w1e = jnp.broadcast_to(
      w1t.reshape(NUM_FIELDS, 1, EMBED_DIM, LAYER1),
      (NUM_FIELDS, 2 * SEG, EMBED_DIM, LAYER1)).reshape(
          NUM_FIELDS * 256, LAYER1)

  bias = (wide_b[0] + bout[0]).reshape(1, 1)
  out = _tc_mlp(
      xs[0], xs[1], seg, dense_input,
      w1e, b1.reshape(1, -1),
      W2.T, b2.reshape(1, -1),
      W3.T, b3.reshape(1, -1),
      Wout, wide_W, bias,
  )
  return out.reshape(BATCH)


# speedup vs baseline: 9.2230x; 1.0022x over previous
import functools

import jax
import jax.numpy as jnp
from jax.experimental import pallas as pl
from jax.experimental.pallas import tpu as pltpu
from jax.experimental.pallas import tpu_sc as plsc

NUM_FIELDS = 26
VOCAB = 100000
EMBED_DIM = 32
BATCH = 4096
SEG = 128 // EMBED_DIM
WINDOW = 128

GROUP_SIZES = (9, 9, 8)
GROUP_STARTS = (0, 9, 18)
LAYER1 = 512

BB = 512
NUM_BB = BATCH // BB

Q_BLK = 12800
FIELD_Q = 25600
NQ = FIELD_Q // Q_BLK
V_NBLK = -(-VOCAB // Q_BLK)
HALF = Q_BLK // 2
PHY_F = FIELD_Q // 2

_VECTOR_MESH = plsc.VectorSubcoreMesh(
    core_axis_name="core", subcore_axis_name="subcore")


def _repack_body(x0_ref, x1_ref, x2_ref, x3_ref, out_ref):
  xs = jnp.concatenate(
      [x0_ref[0], x1_ref[0], x2_ref[0], x3_ref[0]], axis=0)
  t = xs.T
  bits = jax.lax.bitcast_convert_type(t, jnp.int32)
  rb = bits + 0x7FFF + (jax.lax.shift_right_logical(bits, 16) & 1)
  hi16 = jax.lax.shift_right_logical(rb, 16)
  half = Q_BLK // 2
  out_ref[...] = jax.lax.shift_left(hi16[:half], 16) | hi16[half:]


def _tc_repack(bt, f0, gf):
  def vspec(s):
    return pl.BlockSpec(
        (1, EMBED_DIM, Q_BLK),
        lambda f, j, s=s: (f0 + f, 0, jnp.minimum(s * NQ + j, V_NBLK - 1)))

  return pl.pallas_call(
      _repack_body,
      grid=(gf, NQ),
      in_specs=[vspec(0), vspec(1), vspec(2), vspec(3)],
      out_specs=pl.BlockSpec((Q_BLK // 2, 128), lambda f, j: (f * NQ + j, 0)),
      out_shape=jax.ShapeDtypeStruct((gf * PHY_F, 128), jnp.int32),
  )(bt, bt, bt, bt)


def _sc_gather(table128, super_idx, gf):
  nb = BATCH // WINDOW

  @functools.partial(
      pl.kernel,
      out_type=jax.ShapeDtypeStruct((gf, BATCH, 128), jnp.int32),
      mesh=_VECTOR_MESH,
      compiler_params=pltpu.CompilerParams(use_tc_tiling_on_sc=True),
  )
  def gather_kernel(table_hbm, idx_hbm, out_hbm):
    def body(i_vmem, o_vmem):
      pltpu.sync_copy(table_hbm.at[i_vmem.at[0]], o_vmem.at[0])

    pltpu.emit_pipeline(
        body,
        grid=(BATCH * gf // WINDOW,),
        in_specs=[pl.BlockSpec((1, WINDOW), lambda i: (0, i))],
        out_specs=[pl.BlockSpec(
            (1, WINDOW, 128), lambda i: (i // nb, i % nb, 0))],
        core_axis_name=("core", "subcore"),
        dimension_semantics=(pltpu.PARALLEL,),
    )(idx_hbm, out_hbm)

  return gather_kernel(table128, super_idx)


def _mlp_body(x0_ref, x1_ref, x2_ref, seg_ref, d_ref, w1_ref, b1_ref,
              w2_ref, b2_ref, w3_ref, b3_ref, wo_ref, ww_ref, bias_ref,
              o_ref):
  prec = jax.lax.Precision.DEFAULT
  seg = seg_ref[...]
  lane_seg = jax.lax.broadcasted_iota(jnp.int32, (BB, 256), 1) // EMBED_DIM
  h = jnp.zeros((BB, LAYER1), jnp.float32)
  for f0, x_ref in ((0, x0_ref), (9, x1_ref), (18, x2_ref)):
    for f in range(x_ref.shape[0]):
      fg = f0 + f
      xi = x_ref[f]
      hi = jax.lax.bitcast_convert_type(xi & jnp.int32(-65536), jnp.float32)
      lo = jax.lax.bitcast_convert_type(
          jax.lax.shift_left(xi, 16), jnp.float32)
      xf = jnp.concatenate([hi, lo], axis=1).astype(jnp.bfloat16)
      xf = jnp.where(seg[:, fg][:, None] == lane_seg, xf, jnp.bfloat16(0.0))
      h = h + jnp.dot(xf, w1_ref[pl.ds(fg * 256, 256), :],
                      preferred_element_type=jnp.float32, precision=prec)
  h = jnp.maximum(h + b1_ref[...], 0.0)
  h = jnp.dot(h, w2_ref[...], preferred_element_type=jnp.float32,
              precision=prec)
  h = jnp.maximum(h + b2_ref[...], 0.0)
  h = jnp.dot(h, w3_ref[...], preferred_element_type=jnp.float32,
              precision=prec)
  h = jnp.maximum(h + b3_ref[...], 0.0)
  deep = jnp.sum(h * wo_ref[...], axis=1)
  wide = jnp.sum(d_ref[...] * ww_ref[...], axis=1)
  z = 0.5 * (deep + wide + bias_ref[0, 0])
  o_ref[0, 0, :] = jax.nn.sigmoid(z)


def _tc_mlp(xs, seg, dense, w1e, b1, w2t, b2, w3t, b3, wout_row,
            wide_row, bias):
  wspec = lambda shape: pl.BlockSpec(shape, lambda i: tuple(0 for _ in shape))
  xspec = lambda gf: pl.BlockSpec((gf, BB, 128), lambda i: (0, i, 0))
  return pl.pallas_call(
      _mlp_body,
      grid=(NUM_BB,),
      in_specs=[
          xspec(GROUP_SIZES[0]), xspec(GROUP_SIZES[1]), xspec(GROUP_SIZES[2]),
          pl.BlockSpec((BB, NUM_FIELDS), lambda i: (i, 0)),
          pl.BlockSpec((BB, 13), lambda i: (i, 0)),
          wspec(w1e.shape), wspec(b1.shape),
          wspec(w2t.shape), wspec(b2.shape),
          wspec(w3t.shape), wspec(b3.shape),
          wspec(wout_row.shape), wspec(wide_row.shape), wspec(bias.shape),
      ],
      out_specs=pl.BlockSpec((1, 1, BB), lambda i: (i, 0, 0)),
      out_shape=jax.ShapeDtypeStruct((NUM_BB, 1, BB), jnp.float32),
  )(*xs, seg, dense, w1e, b1, w2t, b2, w3t, b3, wout_row, wide_row, bias)


def kernel(dense_input, sparse_input, embed_tables, wide_W, wide_b,
           W1, b1, W2, b2, W3, b3, Wout, bout):
  bt = jnp.transpose(embed_tables, (0, 2, 1))
  sp = sparse_input.astype(jnp.int32)
  spt = sp.T
  kk = sp % FIELD_Q
  seg = (kk % Q_BLK) // HALF * SEG + sp // FIELD_Q
  kt = spt % FIELD_Q
  row_t = (kt // Q_BLK) * HALF + (kt % Q_BLK) % HALF

  xs = []
  for f0, gf in zip(GROUP_STARTS, GROUP_SIZES):
    table_g = _tc_repack(bt, f0, gf)
    offs = (jnp.arange(gf, dtype=jnp.int32) * PHY_F)[:, None]
    idx_g = (row_t[f0:f0 + gf] + offs).reshape(1, BATCH * gf)
    xs.append(_sc_gather(table_g, idx_g, gf))

  w1t = W1.T.astype(jnp.bfloat16)
  w1e = jnp.broadcast_to(
      w1t.reshape(NUM_FIELDS, 1, EMBED_DIM, LAYER1),
      (NUM_FIELDS, 2 * SEG, EMBED_DIM, LAYER1)).reshape(
          NUM_FIELDS * 256, LAYER1)

  bias = (wide_b[0] + bout[0]).reshape(1, 1)
  out = _tc_mlp(
      xs, seg, dense_input,
      w1e, b1.reshape(1, -1),
      W2.T, b2.reshape(1, -1),
      W3.T, b3.reshape(1, -1),
      Wout, wide_W, bias,
  )
  return out.reshape(BATCH)
